# Initial kernel scaffold; baseline (speedup 1.0000x reference)
import jax
import jax.numpy as jnp
from jax.experimental import pallas as pl


def kernel(x, edge_index, batch, W1, b1, W3, b3, W4, b4):
    raise NotImplementedError("write your pallas kernel here")


# speedup vs baseline: 16.0692x; 16.0692x over previous
import functools

import jax
import jax.numpy as jnp
from jax import lax
from jax.experimental import pallas as pl
from jax.experimental.pallas import tpu as pltpu
from jax.experimental.pallas import tpu_sc as plsc

N = 10000
E = 150000
NC = 2
NS = 16
CH = 64
R = 75
EPAD = NC * NS * R * CH
NPAD = 10240
RPT = NPAD // NS

def _get_mesh():
    return plsc.VectorSubcoreMesh(
        core_axis_name="c", subcore_axis_name="s",
        num_cores=NC, num_subcores=NS)


def _make_deg_kernel():
    @functools.partial(
        pl.kernel,
        out_type=jax.ShapeDtypeStruct((NC * NPAD,), jnp.float32),
        mesh=_get_mesh(),
        scratch_types=[
            pltpu.VMEM((R, CH), jnp.int32),
            pltpu.VMEM((CH,), jnp.float32),
            pltpu.VMEM_SHARED((NPAD,), jnp.float32),
        ],
    )
    def deg_kernel(dst2, ones_in, zeros_in, out, idx_d, ones_v, acc):
        c = lax.axis_index("c")
        s = lax.axis_index("s")
        tid = c * NS + s
        pltpu.sync_copy(zeros_in, acc.at[pl.ds(s * RPT, RPT)])
        pltpu.sync_copy(ones_in, ones_v)
        pltpu.sync_copy(dst2.at[tid], idx_d)
        plsc.subcore_barrier()

        def body(j, carry):
            pltpu.sync_copy(ones_v, acc.at[idx_d.at[j]], add=True)
            return carry

        lax.fori_loop(0, R, body, 0)
        plsc.subcore_barrier()
        pltpu.sync_copy(acc.at[pl.ds(s * RPT, RPT)],
                        out.at[pl.ds(c * NPAD + s * RPT, RPT)])

    return deg_kernel


def _make_agg_kernel(F):
    @functools.partial(
        pl.kernel,
        out_type=jax.ShapeDtypeStruct((NC * NPAD, F), jnp.float32),
        mesh=_get_mesh(),
        scratch_types=[
            pltpu.VMEM((R, CH), jnp.int32),
            pltpu.VMEM((R, CH), jnp.int32),
            pltpu.VMEM((CH, F), jnp.float32),
            pltpu.VMEM((CH, F), jnp.float32),
            pltpu.VMEM((CH, F), jnp.float32),
            pltpu.VMEM_SHARED((NPAD, F), jnp.float32),
        ] + [pltpu.SemaphoreType.DMA] * 3,
    )
    def agg_kernel(hs, src2, dst2, zeros_in, out, idx_s, idx_d, rows0, rows1,
                   rows2, acc, sem_a, sem_b, sem_c):
        c = lax.axis_index("c")
        s = lax.axis_index("s")
        tid = c * NS + s
        pltpu.sync_copy(zeros_in, acc.at[pl.ds(s * RPT, RPT)])
        pltpu.sync_copy(src2.at[tid], idx_s)
        pltpu.sync_copy(dst2.at[tid], idx_d)
        plsc.subcore_barrier()

        bufs = (rows0, rows1, rows2)
        sems = (sem_a, sem_b, sem_c)
        pltpu.async_copy(hs.at[idx_s.at[0]], rows0, sem_a)
        pltpu.async_copy(hs.at[idx_s.at[1]], rows1, sem_b)
        nit = R // 3

        def outer(k, carry):
            j0 = 3 * k
            for b in range(3):
                j = j0 + b
                pltpu.async_copy(hs.at[idx_s.at[j + 2]], bufs[(b + 2) % 3],
                                 sems[(b + 2) % 3])
                pltpu.make_async_copy(hs.at[idx_s.at[j]], bufs[b],
                                      sems[b]).wait()
                pltpu.sync_copy(bufs[b], acc.at[idx_d.at[j]], add=True)
            return carry

        lax.fori_loop(0, nit - 1, outer, 0)
        j0 = R - 3
        pltpu.async_copy(hs.at[idx_s.at[j0 + 2]], bufs[(j0 + 2) % 3],
                         sems[(j0 + 2) % 3])
        for b in range(3):
            j = j0 + b
            pltpu.make_async_copy(hs.at[idx_s.at[j]], bufs[(j) % 3],
                                  sems[(j) % 3]).wait()
            pltpu.sync_copy(bufs[(j) % 3], acc.at[idx_d.at[j]], add=True)
        plsc.subcore_barrier()
        pltpu.sync_copy(acc.at[pl.ds(s * RPT, RPT)],
                        out.at[pl.ds(c * NPAD + s * RPT, RPT)])

    return agg_kernel


_sc_cache = {}


def _deg_call(*args):
    if "deg" not in _sc_cache:
        _sc_cache["deg"] = _make_deg_kernel()
    return _sc_cache["deg"](*args)


def _agg_call(F, *args):
    if F not in _sc_cache:
        _sc_cache[F] = _make_agg_kernel(F)
    return _sc_cache[F](*args)

_BLK = 1000
_GRID = N // _BLK


def _mm1_body(degp_ref, x_ref, w_ref, hs_ref, dis_ref):
    dis = lax.rsqrt(degp_ref[0] + degp_ref[1] + 1.0)
    hs_ref[...] = dis * jnp.dot(x_ref[...], w_ref[...],
                                preferred_element_type=jnp.float32)
    dis_ref[...] = dis


def _mm1(degp, x, w1):
    return pl.pallas_call(
        _mm1_body,
        grid=(_GRID,),
        in_specs=[
            pl.BlockSpec((2, _BLK, 1), lambda i: (0, i, 0)),
            pl.BlockSpec((_BLK, 512), lambda i: (i, 0)),
            pl.BlockSpec((512, 128), lambda i: (0, 0)),
        ],
        out_specs=[
            pl.BlockSpec((_BLK, 128), lambda i: (i, 0)),
            pl.BlockSpec((_BLK, 1), lambda i: (i, 0)),
        ],
        out_shape=[
            jax.ShapeDtypeStruct((NPAD, 128), jnp.float32),
            jax.ShapeDtypeStruct((NPAD, 1), jnp.float32),
        ],
    )(degp, x, w1)


def _mid_body(dis_ref, agg_ref, hs_ref, w_ref, b_ref, o_ref):
    dis = dis_ref[...]
    h = jnp.maximum(dis * (agg_ref[0] + agg_ref[1] + hs_ref[...]) + b_ref[...],
                    0.0)
    o_ref[...] = dis * jnp.dot(h, w_ref[...],
                               preferred_element_type=jnp.float32)


def _mid(dis, agg, hs, w, b, fin, fout):
    return pl.pallas_call(
        _mid_body,
        grid=(_GRID,),
        in_specs=[
            pl.BlockSpec((_BLK, 1), lambda i: (i, 0)),
            pl.BlockSpec((2, _BLK, fin), lambda i: (0, i, 0)),
            pl.BlockSpec((_BLK, fin), lambda i: (i, 0)),
            pl.BlockSpec((fin, fout), lambda i: (0, 0)),
            pl.BlockSpec((1, fin), lambda i: (0, 0)),
        ],
        out_specs=pl.BlockSpec((_BLK, fout), lambda i: (i, 0)),
        out_shape=jax.ShapeDtypeStruct((NPAD, fout), jnp.float32),
    )(dis, agg, hs, w, b)


def _fin_body(dis_ref, agg_ref, p_ref, b_ref, batch_ref, o_ref, g_ref, c_ref):
    i = pl.program_id(0)
    out3 = (dis_ref[...] * (agg_ref[0] + agg_ref[1] + p_ref[...])
            + b_ref[...])[:, 0:2]
    seg = (batch_ref[...] == lax.broadcasted_iota(jnp.int32, (1, 64), 1)
           ).astype(jnp.float32)
    g_blk = lax.dot_general(seg, out3, (((0,), (0,)), ((), ())),
                            preferred_element_type=jnp.float32)
    c_blk = lax.dot_general(seg, jnp.ones_like(out3), (((0,), (0,)), ((), ())),
                            preferred_element_type=jnp.float32)

    @pl.when(i == 0)
    def _():
        g_ref[...] = jnp.zeros_like(g_ref)
        c_ref[...] = jnp.zeros_like(c_ref)

    g_ref[...] += g_blk
    c_ref[...] += c_blk

    @pl.when(i == _GRID - 1)
    def _():
        lg = g_ref[...] / jnp.maximum(c_ref[...], 1.0)
        m = jnp.max(lg, axis=1, keepdims=True)
        ls = lg - m - jnp.log(jnp.sum(jnp.exp(lg - m), axis=1, keepdims=True))
        o_ref[...] = jnp.concatenate(
            [ls, jnp.zeros((64, 14), jnp.float32)], axis=1)


def _final(dis, agg, p, b4p, batch2d):
    return pl.pallas_call(
        _fin_body,
        grid=(_GRID,),
        in_specs=[
            pl.BlockSpec((_BLK, 1), lambda i: (i, 0)),
            pl.BlockSpec((2, _BLK, 128), lambda i: (0, i, 0)),
            pl.BlockSpec((_BLK, 128), lambda i: (i, 0)),
            pl.BlockSpec((1, 128), lambda i: (0, 0)),
            pl.BlockSpec((_BLK, 1), lambda i: (i, 0)),
        ],
        out_specs=pl.BlockSpec((64, 16), lambda i: (0, 0)),
        out_shape=jax.ShapeDtypeStruct((64, 16), jnp.float32),
        scratch_shapes=[
            pltpu.VMEM((64, 2), jnp.float32),
            pltpu.VMEM((64, 2), jnp.float32),
        ],
    )(dis, agg, p, b4p, batch2d)


def kernel(x, edge_index, batch, W1, b1, W3, b3, W4, b4):
    ei = edge_index.astype(jnp.int32)
    pad_iota = jnp.arange(EPAD - E, dtype=jnp.int32)
    pad_src = pad_iota % N
    pad_dst = N + pad_iota % (NPAD - N)
    src2 = jnp.concatenate([ei[0], pad_src]).reshape(NC * NS, R, CH)
    dst2 = jnp.concatenate([ei[1], pad_dst]).reshape(NC * NS, R, CH)
    batch2d = batch.astype(jnp.int32).reshape(N, 1)

    ones_in = jnp.ones((CH,), jnp.float32)
    z1 = jnp.zeros((RPT,), jnp.float32)
    z128 = jnp.zeros((RPT, 128), jnp.float32)

    degp = _deg_call(dst2, ones_in, z1).reshape(NC, NPAD, 1)
    hs1, dis = _mm1(degp, x, W1)

    agg1 = _agg_call(128, hs1, src2, dst2, z128).reshape(NC, NPAD, 128)
    w3p = jnp.pad(W3, ((0, 0), (0, 96)))
    hs2 = _mid(dis, agg1, hs1, w3p, b1.reshape(1, 128), 128, 128)

    agg2 = _agg_call(128, hs2, src2, dst2, z128).reshape(NC, NPAD, 128)
    w4p = jnp.pad(W4, ((0, 96), (0, 126)))
    b3p = jnp.pad(b3, (0, 96)).reshape(1, 128)
    p = _mid(dis, agg2, hs2, w4p, b3p, 128, 128)

    agg3 = _agg_call(128, p, src2, dst2, z128).reshape(NC, NPAD, 128)
    b4p = jnp.pad(b4, (0, 126)).reshape(1, 128)
    out = _final(dis, agg3, p, b4p, batch2d)
    return out[:, :2]


# speedup vs baseline: 17.2208x; 1.0717x over previous
import functools

import jax
import jax.numpy as jnp
from jax import lax
from jax.experimental import pallas as pl
from jax.experimental.pallas import tpu as pltpu
from jax.experimental.pallas import tpu_sc as plsc

N = 10000
E = 150000
NC = 2
NS = 16
CH = 64
R = 75
EPAD = NC * NS * R * CH
NPAD = 10240
RPT = NPAD // NS

def _get_mesh():
    return plsc.VectorSubcoreMesh(
        core_axis_name="c", subcore_axis_name="s",
        num_cores=NC, num_subcores=NS)


def _make_deg_kernel():
    @functools.partial(
        pl.kernel,
        out_type=jax.ShapeDtypeStruct((NC * NPAD,), jnp.float32),
        mesh=_get_mesh(),
        scratch_types=[
            pltpu.VMEM((R, CH), jnp.int32),
            pltpu.VMEM((CH,), jnp.float32),
            pltpu.VMEM((CH,), jnp.float32),
            pltpu.VMEM_SHARED((NPAD,), jnp.float32),
        ],
    )
    def deg_kernel(dst2, out, idx_d, ones_v, zbuf, acc):
        c = lax.axis_index("c")
        s = lax.axis_index("s")
        tid = c * NS + s
        for k in range(CH // 16):
            ones_v[pl.ds(16 * k, 16)] = jnp.ones((16,), jnp.float32)
            zbuf[pl.ds(16 * k, 16)] = jnp.zeros((16,), jnp.float32)

        def zcopy(t, carry):
            pltpu.sync_copy(zbuf, acc.at[pl.ds(s * RPT + t * CH, CH)])
            return carry

        lax.fori_loop(0, RPT // CH, zcopy, 0)
        pltpu.sync_copy(dst2.at[tid], idx_d)
        plsc.subcore_barrier()

        def body(j, carry):
            pltpu.sync_copy(ones_v, acc.at[idx_d.at[j]], add=True)
            return carry

        lax.fori_loop(0, R, body, 0)
        plsc.subcore_barrier()
        pltpu.sync_copy(acc.at[pl.ds(s * RPT, RPT)],
                        out.at[pl.ds(c * NPAD + s * RPT, RPT)])

    return deg_kernel


def _make_agg_kernel(F):
    @functools.partial(
        pl.kernel,
        out_type=jax.ShapeDtypeStruct((NC * NPAD, F), jnp.float32),
        mesh=_get_mesh(),
        scratch_types=[
            pltpu.VMEM((R, CH), jnp.int32),
            pltpu.VMEM((R, CH), jnp.int32),
            pltpu.VMEM((CH, F), jnp.float32),
            pltpu.VMEM((CH, F), jnp.float32),
            pltpu.VMEM((CH, F), jnp.float32),
            pltpu.VMEM_SHARED((NPAD, F), jnp.float32),
        ] + [pltpu.SemaphoreType.DMA] * 3,
    )
    def agg_kernel(hs, src2, dst2, out, idx_s, idx_d, rows0, rows1,
                   rows2, acc, sem_a, sem_b, sem_c):
        c = lax.axis_index("c")
        s = lax.axis_index("s")
        tid = c * NS + s

        zero16 = jnp.zeros((16,), jnp.float32)

        def zrow(r, carry):
            for k in range(F // 16):
                rows0[r, pl.ds(16 * k, 16)] = zero16
            return carry

        lax.fori_loop(0, CH, zrow, 0)

        def zcopy(t, carry):
            pltpu.sync_copy(rows0, acc.at[pl.ds(s * RPT + t * CH, CH)])
            return carry

        lax.fori_loop(0, RPT // CH, zcopy, 0)
        pltpu.sync_copy(src2.at[tid], idx_s)
        pltpu.sync_copy(dst2.at[tid], idx_d)
        plsc.subcore_barrier()

        bufs = (rows0, rows1, rows2)
        sems = (sem_a, sem_b, sem_c)
        pltpu.async_copy(hs.at[idx_s.at[0]], rows0, sem_a)
        pltpu.async_copy(hs.at[idx_s.at[1]], rows1, sem_b)
        nit = R // 3

        def outer(k, carry):
            j0 = 3 * k
            for b in range(3):
                j = j0 + b
                pltpu.async_copy(hs.at[idx_s.at[j + 2]], bufs[(b + 2) % 3],
                                 sems[(b + 2) % 3])
                pltpu.make_async_copy(hs.at[idx_s.at[j]], bufs[b],
                                      sems[b]).wait()
                pltpu.sync_copy(bufs[b], acc.at[idx_d.at[j]], add=True)
            return carry

        lax.fori_loop(0, nit - 1, outer, 0)
        j0 = R - 3
        pltpu.async_copy(hs.at[idx_s.at[j0 + 2]], bufs[(j0 + 2) % 3],
                         sems[(j0 + 2) % 3])
        for b in range(3):
            j = j0 + b
            pltpu.make_async_copy(hs.at[idx_s.at[j]], bufs[(j) % 3],
                                  sems[(j) % 3]).wait()
            pltpu.sync_copy(bufs[(j) % 3], acc.at[idx_d.at[j]], add=True)
        plsc.subcore_barrier()
        pltpu.sync_copy(acc.at[pl.ds(s * RPT, RPT)],
                        out.at[pl.ds(c * NPAD + s * RPT, RPT)])

    return agg_kernel


_sc_cache = {}


def _deg_call(*args):
    if "deg" not in _sc_cache:
        _sc_cache["deg"] = _make_deg_kernel()
    return _sc_cache["deg"](*args)


def _agg_call(F, *args):
    if F not in _sc_cache:
        _sc_cache[F] = _make_agg_kernel(F)
    return _sc_cache[F](*args)

_BLK = 1000
_GRID = N // _BLK


def _mm1_body(degp_ref, x_ref, w_ref, hs_ref, dis_ref):
    dis = lax.rsqrt(degp_ref[0] + degp_ref[1] + 1.0)
    hs_ref[...] = dis * jnp.dot(x_ref[...], w_ref[...],
                                preferred_element_type=jnp.float32)
    dis_ref[...] = dis


def _mm1(degp, x, w1):
    return pl.pallas_call(
        _mm1_body,
        grid=(_GRID,),
        in_specs=[
            pl.BlockSpec((2, _BLK, 1), lambda i: (0, i, 0)),
            pl.BlockSpec((_BLK, 512), lambda i: (i, 0)),
            pl.BlockSpec((512, 128), lambda i: (0, 0)),
        ],
        out_specs=[
            pl.BlockSpec((_BLK, 128), lambda i: (i, 0)),
            pl.BlockSpec((_BLK, 1), lambda i: (i, 0)),
        ],
        out_shape=[
            jax.ShapeDtypeStruct((NPAD, 128), jnp.float32),
            jax.ShapeDtypeStruct((NPAD, 1), jnp.float32),
        ],
    )(degp, x, w1)


def _mid_body(dis_ref, agg_ref, hs_ref, w_ref, b_ref, o_ref):
    dis = dis_ref[...]
    h = jnp.maximum(dis * (agg_ref[0] + agg_ref[1] + hs_ref[...]) + b_ref[...],
                    0.0)
    o_ref[...] = dis * jnp.dot(h, w_ref[...],
                               preferred_element_type=jnp.float32)


def _mid(dis, agg, hs, w, b, fin, fout):
    return pl.pallas_call(
        _mid_body,
        grid=(_GRID,),
        in_specs=[
            pl.BlockSpec((_BLK, 1), lambda i: (i, 0)),
            pl.BlockSpec((2, _BLK, fin), lambda i: (0, i, 0)),
            pl.BlockSpec((_BLK, fin), lambda i: (i, 0)),
            pl.BlockSpec((fin, fout), lambda i: (0, 0)),
            pl.BlockSpec((1, fin), lambda i: (0, 0)),
        ],
        out_specs=pl.BlockSpec((_BLK, fout), lambda i: (i, 0)),
        out_shape=jax.ShapeDtypeStruct((NPAD, fout), jnp.float32),
    )(dis, agg, hs, w, b)


def _fin_body(dis_ref, agg_ref, p_ref, b_ref, batch_ref, o_ref, g_ref, c_ref):
    i = pl.program_id(0)
    out3 = (dis_ref[...] * (agg_ref[0] + agg_ref[1] + p_ref[...])
            + b_ref[...])[:, 0:2]
    seg = (batch_ref[...] == lax.broadcasted_iota(jnp.int32, (1, 64), 1)
           ).astype(jnp.float32)
    g_blk = lax.dot_general(seg, out3, (((0,), (0,)), ((), ())),
                            preferred_element_type=jnp.float32)
    c_blk = lax.dot_general(seg, jnp.ones_like(out3), (((0,), (0,)), ((), ())),
                            preferred_element_type=jnp.float32)

    @pl.when(i == 0)
    def _():
        g_ref[...] = jnp.zeros_like(g_ref)
        c_ref[...] = jnp.zeros_like(c_ref)

    g_ref[...] += g_blk
    c_ref[...] += c_blk

    @pl.when(i == _GRID - 1)
    def _():
        lg = g_ref[...] / jnp.maximum(c_ref[...], 1.0)
        m = jnp.max(lg, axis=1, keepdims=True)
        ls = lg - m - jnp.log(jnp.sum(jnp.exp(lg - m), axis=1, keepdims=True))
        o_ref[...] = jnp.concatenate(
            [ls, jnp.zeros((64, 14), jnp.float32)], axis=1)


def _final(dis, agg, p, b4p, batch2d):
    return pl.pallas_call(
        _fin_body,
        grid=(_GRID,),
        in_specs=[
            pl.BlockSpec((_BLK, 1), lambda i: (i, 0)),
            pl.BlockSpec((2, _BLK, 128), lambda i: (0, i, 0)),
            pl.BlockSpec((_BLK, 128), lambda i: (i, 0)),
            pl.BlockSpec((1, 128), lambda i: (0, 0)),
            pl.BlockSpec((_BLK, 1), lambda i: (i, 0)),
        ],
        out_specs=pl.BlockSpec((64, 16), lambda i: (0, 0)),
        out_shape=jax.ShapeDtypeStruct((64, 16), jnp.float32),
        scratch_shapes=[
            pltpu.VMEM((64, 2), jnp.float32),
            pltpu.VMEM((64, 2), jnp.float32),
        ],
    )(dis, agg, p, b4p, batch2d)


def kernel(x, edge_index, batch, W1, b1, W3, b3, W4, b4):
    ei = edge_index.astype(jnp.int32)
    pad_iota = jnp.arange(EPAD - E, dtype=jnp.int32)
    pad_src = pad_iota % N
    pad_dst = N + pad_iota % (NPAD - N)
    src2 = jnp.concatenate([ei[0], pad_src]).reshape(NC * NS, R, CH)
    dst2 = jnp.concatenate([ei[1], pad_dst]).reshape(NC * NS, R, CH)
    batch2d = batch.astype(jnp.int32).reshape(N, 1)

    degp = _deg_call(dst2).reshape(NC, NPAD, 1)
    hs1, dis = _mm1(degp, x, W1)

    agg1 = _agg_call(128, hs1, src2, dst2).reshape(NC, NPAD, 128)
    w3p = jnp.pad(W3, ((0, 0), (0, 96)))
    hs2 = _mid(dis, agg1, hs1, w3p, b1.reshape(1, 128), 128, 128)

    agg2 = _agg_call(128, hs2, src2, dst2).reshape(NC, NPAD, 128)
    w4p = jnp.pad(W4, ((0, 96), (0, 126)))
    b3p = jnp.pad(b3, (0, 96)).reshape(1, 128)
    p = _mid(dis, agg2, hs2, w4p, b3p, 128, 128)

    agg3 = _agg_call(128, p, src2, dst2).reshape(NC, NPAD, 128)
    b4p = jnp.pad(b4, (0, 126)).reshape(1, 128)
    out = _final(dis, agg3, p, b4p, batch2d)
    return out[:, :2]
